# baseline (speedup 1.0000x reference)
import jax
import jax.numpy as jnp
import numpy as np
from jax import lax
from jax.experimental import pallas as pl
from jax.experimental.pallas import tpu as pltpu
from jax.experimental.pallas import tpu_sc as plsc

VOCAB_SIZE = 100000
D_MODEL = 768
MAX_LEN = 4096
BATCH = 4
SEQ_LEN = 4096

NC = 2
NS = 16
NW = NC * NS
POS_PER_W = SEQ_LEN // NW
P = 32
N_CHUNK = POS_PER_W // P
LANES = 16


def _sinusoidal_pos_encoding(max_len, d_model):
    pos = np.arange(max_len, dtype=np.float32)[:, None]
    i = np.arange(0, d_model, 2, dtype=np.float32)[None, :]
    angle = pos / np.power(10000.0, i / d_model)
    enc = np.zeros((max_len, d_model), dtype=np.float32)
    enc[:, 0::2] = np.sin(angle)
    enc[:, 1::2] = np.cos(angle)
    return enc


_POS_ENC = _sinusoidal_pos_encoding(MAX_LEN, D_MODEL)


def _embed_kernel(tab_hbm, idx_hbm, pos_hbm, out_hbm, idx_v, pos_v, tok_v, sem):
    wid = lax.axis_index("s") * NC + lax.axis_index("c")
    pos_base = wid * POS_PER_W

    def add_row(r, carry):
        for j in range(D_MODEL // LANES):
            sl = pl.ds(j * LANES, LANES)
            tok_v[r, sl] = tok_v[r, sl] + pos_v[r, sl]
        return carry

    for c in range(N_CHUNK):
        pbase = pos_base + c * P
        pltpu.sync_copy(pos_hbm.at[pl.ds(pbase, P)], pos_v)
        for b in range(BATCH):
            row0 = b * SEQ_LEN + pbase
            pltpu.sync_copy(idx_hbm.at[pl.ds(row0, P)], idx_v)
            pltpu.async_copy(tab_hbm.at[idx_v], tok_v, sem).wait()
            lax.fori_loop(0, P, add_row, 0)
            pltpu.sync_copy(tok_v, out_hbm.at[pl.ds(row0, P)])


@jax.jit
def _embed(x_flat, tok_table, pos_enc):
    mesh = plsc.VectorSubcoreMesh(core_axis_name="c", subcore_axis_name="s")
    run = pl.kernel(
        _embed_kernel,
        out_type=jax.ShapeDtypeStruct((BATCH * SEQ_LEN, D_MODEL), jnp.float32),
        mesh=mesh,
        scratch_types=[
            pltpu.VMEM((P,), jnp.int32),
            pltpu.VMEM((P, D_MODEL), jnp.float32),
            pltpu.VMEM((P, D_MODEL), jnp.float32),
            pltpu.SemaphoreType.DMA,
        ],
    )
    return run(tok_table, x_flat, pos_enc)


def kernel(x, tok_table):
    x_flat = x.reshape(-1).astype(jnp.int32)
    out = _embed(x_flat, tok_table, _POS_ENC)
    return out.reshape(BATCH, SEQ_LEN, D_MODEL)


# speedup vs baseline: 1.4246x; 1.4246x over previous
import jax
import jax.numpy as jnp
import numpy as np
from jax import lax
from jax.experimental import pallas as pl
from jax.experimental.pallas import tpu as pltpu
from jax.experimental.pallas import tpu_sc as plsc

VOCAB_SIZE = 100000
D_MODEL = 768
MAX_LEN = 4096
BATCH = 4
SEQ_LEN = 4096

NC = 2
NS = 16
NW = NC * NS
POS_PER_W = SEQ_LEN // NW
P = 32
N_CHUNK = POS_PER_W // P
N_TASK = N_CHUNK * BATCH
LANES = 16


def _sinusoidal_pos_encoding(max_len, d_model):
    pos = np.arange(max_len, dtype=np.float32)[:, None]
    i = np.arange(0, d_model, 2, dtype=np.float32)[None, :]
    angle = pos / np.power(10000.0, i / d_model)
    enc = np.zeros((max_len, d_model), dtype=np.float32)
    enc[:, 0::2] = np.sin(angle)
    enc[:, 1::2] = np.cos(angle)
    return enc


_POS_ENC = _sinusoidal_pos_encoding(MAX_LEN, D_MODEL)


def _embed_kernel(tab_hbm, idx_hbm, pos_hbm, out_hbm,
                  idx_v, pos0, pos1, tok0, tok1,
                  gsem0, gsem1, ssem0, ssem1, psem):
    wid = lax.axis_index("s") * NC + lax.axis_index("c")
    pos_base = wid * POS_PER_W

    toks = [tok0, tok1]
    gsems = [gsem0, gsem1]
    ssems = [ssem0, ssem1]
    poss = [pos0, pos1]

    icp = []
    for b in range(BATCH):
        icp.append(pltpu.async_copy(
            idx_hbm.at[pl.ds(b * SEQ_LEN + pos_base, POS_PER_W)],
            idx_v.at[pl.ds(b * POS_PER_W, POS_PER_W)], psem))
    for cp in icp:
        cp.wait()

    pltpu.sync_copy(pos_hbm.at[pl.ds(pos_base, P)], pos0)

    def start_gather(t):
        c, b = divmod(t, BATCH)
        isl = idx_v.at[pl.ds(b * POS_PER_W + c * P, P)]
        return pltpu.async_copy(tab_hbm.at[isl], toks[t % 2], gsems[t % 2])

    def add_pos(tok, posb):
        def add_row(r, carry):
            for j in range(D_MODEL // LANES):
                sl = pl.ds(j * LANES, LANES)
                tok[r, sl] = tok[r, sl] + posb[r, sl]
            return carry
        lax.fori_loop(0, P, add_row, 0)

    g_cp = [None] * N_TASK
    s_cp = [None] * N_TASK
    p_cp = [None] * N_CHUNK

    g_cp[0] = start_gather(0)
    for t in range(N_TASK):
        c, b = divmod(t, BATCH)
        if b == 0 and c + 1 < N_CHUNK:
            p_cp[c + 1] = pltpu.async_copy(
                pos_hbm.at[pl.ds(pos_base + (c + 1) * P, P)],
                poss[(c + 1) % 2], psem)
        if t + 1 < N_TASK:
            if t >= 1:
                s_cp[t - 1].wait()
            g_cp[t + 1] = start_gather(t + 1)
        g_cp[t].wait()
        if b == 0 and c > 0:
            p_cp[c].wait()
        add_pos(toks[t % 2], poss[c % 2])
        s_cp[t] = pltpu.async_copy(
            toks[t % 2],
            out_hbm.at[pl.ds(b * SEQ_LEN + pos_base + c * P, P)],
            ssems[t % 2])
    s_cp[N_TASK - 2].wait()
    s_cp[N_TASK - 1].wait()


@jax.jit
def _embed(x_flat, tok_table, pos_enc):
    mesh = plsc.VectorSubcoreMesh(core_axis_name="c", subcore_axis_name="s")
    run = pl.kernel(
        _embed_kernel,
        out_type=jax.ShapeDtypeStruct((BATCH * SEQ_LEN, D_MODEL), jnp.float32),
        mesh=mesh,
        scratch_types=[
            pltpu.VMEM((BATCH * POS_PER_W,), jnp.int32),
            pltpu.VMEM((P, D_MODEL), jnp.float32),
            pltpu.VMEM((P, D_MODEL), jnp.float32),
            pltpu.VMEM((P, D_MODEL), jnp.float32),
            pltpu.VMEM((P, D_MODEL), jnp.float32),
            pltpu.SemaphoreType.DMA,
            pltpu.SemaphoreType.DMA,
            pltpu.SemaphoreType.DMA,
            pltpu.SemaphoreType.DMA,
            pltpu.SemaphoreType.DMA,
        ],
    )
    return run(tok_table, x_flat, pos_enc)


def kernel(x, tok_table):
    x_flat = x.reshape(-1).astype(jnp.int32)
    out = _embed(x_flat, tok_table, _POS_ENC)
    return out.reshape(BATCH, SEQ_LEN, D_MODEL)
